# baseline (speedup 1.0000x reference)
import functools

import jax
import jax.numpy as jnp
from jax import lax
from jax.experimental import pallas as pl
from jax.experimental.pallas import tpu as pltpu
from jax.experimental.pallas import tpu_sc as plsc

N_ATOMS = 10000
N_BONDS = 320000
MAX_NB = 32
ATOM_FDIM = 128
BOND_FDIM = 16
HIDDEN = 128
DEPTH = 4

NC, NS, NL = 2, 16, 16
NW = NC * NS
NVH = HIDDEN // NL

_SC_MESH = plsc.VectorSubcoreMesh(core_axis_name="c", subcore_axis_name="s")

CA = 4
N_CHUNKS_A = N_ATOMS // CA
CI_A = CA * MAX_NB
CM_A = N_CHUNKS_A // NW
EXTRA_A = NW * CM_A


def _sum_chunk(rows_v, out_slab, o_base):
    for a in range(CA):
        accs = tuple(jnp.zeros((NL,), jnp.float32) for _ in range(NVH))

        def red(k2, acc, a=a):
            k = k2 * 4
            for u in range(4):
                acc = tuple(
                    acc[j] + rows_v[a * MAX_NB + k + u, pl.ds(j * NL, NL)]
                    for j in range(NVH))
            return acc

        accs = lax.fori_loop(0, MAX_NB // 4, red, accs)
        for j in range(NVH):
            out_slab[o_base + a, pl.ds(j * NL, NL)] = accs[j]


def _sc_gather_sum_body(msg_hbm, a2b_hbm, out_hbm, idx_v, rows0, rows1,
                        out_slab, xi_v, xo_v, sem0, sem1):
    wid = lax.axis_index("s") * NC + lax.axis_index("c")
    base_c = wid * CM_A
    base_idx = base_c * CI_A

    pltpu.sync_copy(a2b_hbm.at[pl.ds(base_idx, CM_A * CI_A)], idx_v)

    def g(i, rows, sem):
        return pltpu.make_async_copy(
            msg_hbm.at[idx_v.at[pl.ds(i * CI_A, CI_A)]], rows, sem)

    g(0, rows0, sem0).start()

    def pair(t, carry):
        i0 = 2 * t
        g(i0 + 1, rows1, sem1).start()
        g(i0, rows0, sem0).wait()
        _sum_chunk(rows0, out_slab, i0 * CA)

        @pl.when(t < CM_A // 2 - 1)
        def _():
            g(i0 + 2, rows0, sem0).start()

        g(i0 + 1, rows1, sem1).wait()
        _sum_chunk(rows1, out_slab, (i0 + 1) * CA)
        return carry

    lax.fori_loop(0, CM_A // 2, pair, 0)
    pltpu.sync_copy(out_slab, out_hbm.at[pl.ds(base_c * CA, CM_A * CA)])

    @pl.when(wid < N_CHUNKS_A - EXTRA_A)
    def _tail():
        c = EXTRA_A + wid
        pltpu.sync_copy(a2b_hbm.at[pl.ds(c * CI_A, CI_A)], xi_v)
        pltpu.async_copy(msg_hbm.at[xi_v], rows0, sem0).wait()
        _sum_chunk(rows0, xo_v, 0)
        pltpu.sync_copy(xo_v, out_hbm.at[pl.ds(c * CA, CA)])


_sc_gather_sum = functools.partial(
    pl.kernel,
    mesh=_SC_MESH,
    out_type=jax.ShapeDtypeStruct((N_ATOMS, HIDDEN), jnp.float32),
    scratch_types=[
        pltpu.VMEM((CM_A * CI_A,), jnp.int32),
        pltpu.VMEM((CI_A, HIDDEN), jnp.float32),
        pltpu.VMEM((CI_A, HIDDEN), jnp.float32),
        pltpu.VMEM((CM_A * CA, HIDDEN), jnp.float32),
        pltpu.VMEM((CI_A,), jnp.int32),
        pltpu.VMEM((CA, HIDDEN), jnp.float32),
        pltpu.SemaphoreType.DMA,
        pltpu.SemaphoreType.DMA,
    ],
)(_sc_gather_sum_body)


CB = 128
N_CHUNKS_B = N_BONDS // CB
CM_B = N_CHUNKS_B // NW
EXTRA_B = NW * CM_B


def _sub_chunk(rows_a, rows_r, obuf):
    def sub4(q, carry):
        b = q * 4
        for bb in range(4):
            for j in range(NVH):
                obuf[b + bb, pl.ds(j * NL, NL)] = (
                    rows_a[b + bb, pl.ds(j * NL, NL)]
                    - rows_r[b + bb, pl.ds(j * NL, NL)])
        return carry

    lax.fori_loop(0, CB // 4, sub4, 0)


def _sc_gather_diff_body(amsg_hbm, msg_hbm, b2a_hbm, b2revb_hbm, out_hbm,
                         idxa_v, idxr_v, a0, r0, a1, r1, o0, o1, xi_v,
                         sga0, sgr0, sga1, sgr1, so0, so1):
    wid = lax.axis_index("s") * NC + lax.axis_index("c")
    base_c = wid * CM_B
    base_row = base_c * CB

    pltpu.sync_copy(b2a_hbm.at[pl.ds(base_row, CM_B * CB)], idxa_v)
    pltpu.sync_copy(b2revb_hbm.at[pl.ds(base_row, CM_B * CB)], idxr_v)

    def ga(i, rows, sem):
        return pltpu.make_async_copy(
            amsg_hbm.at[idxa_v.at[pl.ds(i * CB, CB)]], rows, sem)

    def gr(i, rows, sem):
        return pltpu.make_async_copy(
            msg_hbm.at[idxr_v.at[pl.ds(i * CB, CB)]], rows, sem)

    def oc(i, obuf, sem):
        return pltpu.make_async_copy(
            obuf, out_hbm.at[pl.ds(base_row + i * CB, CB)], sem)

    ga(0, a0, sga0).start()
    gr(0, r0, sgr0).start()

    def pair(t, carry):
        i0 = 2 * t
        ga(i0 + 1, a1, sga1).start()
        gr(i0 + 1, r1, sgr1).start()
        ga(i0, a0, sga0).wait()
        gr(i0, r0, sgr0).wait()

        @pl.when(t > 0)
        def _():
            oc(i0 - 2, o0, so0).wait()

        _sub_chunk(a0, r0, o0)
        oc(i0, o0, so0).start()

        @pl.when(t < CM_B // 2 - 1)
        def _():
            ga(i0 + 2, a0, sga0).start()
            gr(i0 + 2, r0, sgr0).start()

        ga(i0 + 1, a1, sga1).wait()
        gr(i0 + 1, r1, sgr1).wait()

        @pl.when(t > 0)
        def _():
            oc(i0 - 1, o1, so1).wait()

        _sub_chunk(a1, r1, o1)
        oc(i0 + 1, o1, so1).start()
        return carry

    lax.fori_loop(0, CM_B // 2, pair, 0)
    oc(CM_B - 2, o0, so0).wait()
    oc(CM_B - 1, o1, so1).wait()

    @pl.when(wid < N_CHUNKS_B - EXTRA_B)
    def _tail():
        c = EXTRA_B + wid
        pltpu.sync_copy(b2a_hbm.at[pl.ds(c * CB, CB)], xi_v)
        pltpu.async_copy(amsg_hbm.at[xi_v], a0, sga0).wait()
        pltpu.sync_copy(b2revb_hbm.at[pl.ds(c * CB, CB)], xi_v)
        pltpu.async_copy(msg_hbm.at[xi_v], r0, sgr0).wait()
        _sub_chunk(a0, r0, o0)
        pltpu.sync_copy(o0, out_hbm.at[pl.ds(c * CB, CB)])


_sc_gather_diff = functools.partial(
    pl.kernel,
    mesh=_SC_MESH,
    out_type=jax.ShapeDtypeStruct((N_BONDS, HIDDEN), jnp.float32),
    scratch_types=[
        pltpu.VMEM((CM_B * CB,), jnp.int32),
        pltpu.VMEM((CM_B * CB,), jnp.int32),
        pltpu.VMEM((CB, HIDDEN), jnp.float32),
        pltpu.VMEM((CB, HIDDEN), jnp.float32),
        pltpu.VMEM((CB, HIDDEN), jnp.float32),
        pltpu.VMEM((CB, HIDDEN), jnp.float32),
        pltpu.VMEM((CB, HIDDEN), jnp.float32),
        pltpu.VMEM((CB, HIDDEN), jnp.float32),
        pltpu.VMEM((CB,), jnp.int32),
        pltpu.SemaphoreType.DMA,
        pltpu.SemaphoreType.DMA,
        pltpu.SemaphoreType.DMA,
        pltpu.SemaphoreType.DMA,
        pltpu.SemaphoreType.DMA,
        pltpu.SemaphoreType.DMA,
    ],
)(_sc_gather_diff_body)


BN = 2000
BA = 2000


def _tc_pre_body(fb_ref, wit_ref, inp_ref):
    inp_ref[...] = jnp.dot(fb_ref[...], wit_ref[...],
                           preferred_element_type=jnp.float32)


def _tc_pre(f_bonds, wit):
    return pl.pallas_call(
        _tc_pre_body,
        grid=(N_BONDS // BN,),
        in_specs=[
            pl.BlockSpec((BN, BOND_FDIM), lambda i: (i, 0)),
            pl.BlockSpec((BOND_FDIM, HIDDEN), lambda i: (0, 0)),
        ],
        out_specs=pl.BlockSpec((BN, HIDDEN), lambda i: (i, 0)),
        out_shape=jax.ShapeDtypeStruct((N_BONDS, HIDDEN), jnp.float32),
    )(f_bonds, wit)


def _tc_gru_body(m_ref, inp_ref, wiht_ref, bih_ref, whht_ref, bhh_ref,
                 out_ref):
    m = m_ref[...]
    gh = (jnp.dot(m, whht_ref[...], preferred_element_type=jnp.float32)
          + bhh_ref[...])
    gi = (jnp.dot(inp_ref[...], wiht_ref[...],
                  preferred_element_type=jnp.float32)
          + bih_ref[...])
    r = jax.nn.sigmoid(gi[:, :HIDDEN] + gh[:, :HIDDEN])
    z = jax.nn.sigmoid(gi[:, HIDDEN:2 * HIDDEN] + gh[:, HIDDEN:2 * HIDDEN])
    n = jnp.tanh(gi[:, 2 * HIDDEN:] + r * gh[:, 2 * HIDDEN:])
    out_ref[...] = (1.0 - z) * n + z * m

    @pl.when(pl.program_id(0) == 0)
    def _zero_row0():
        out_ref[0:1, :] = jnp.zeros((1, HIDDEN), jnp.float32)


def _tc_gru(m_in, inp, wiht, bih, whht, bhh):
    return pl.pallas_call(
        _tc_gru_body,
        grid=(N_BONDS // BN,),
        in_specs=[
            pl.BlockSpec((BN, HIDDEN), lambda i: (i, 0)),
            pl.BlockSpec((BN, HIDDEN), lambda i: (i, 0)),
            pl.BlockSpec((HIDDEN, 3 * HIDDEN), lambda i: (0, 0)),
            pl.BlockSpec((1, 3 * HIDDEN), lambda i: (0, 0)),
            pl.BlockSpec((HIDDEN, 3 * HIDDEN), lambda i: (0, 0)),
            pl.BlockSpec((1, 3 * HIDDEN), lambda i: (0, 0)),
        ],
        out_specs=pl.BlockSpec((BN, HIDDEN), lambda i: (i, 0)),
        out_shape=jax.ShapeDtypeStruct((N_BONDS, HIDDEN), jnp.float32),
    )(m_in, inp, wiht, bih, whht, bhh)


def _tc_out_body(fa_ref, am_ref, woa_ref, wom_ref, bo_ref, mask_ref, o_ref):
    h = (jnp.dot(fa_ref[...], woa_ref[...], preferred_element_type=jnp.float32)
         + jnp.dot(am_ref[...], wom_ref[...], preferred_element_type=jnp.float32)
         + bo_ref[...])
    o_ref[...] = jnp.maximum(h, 0.0) * mask_ref[...]


def _tc_out(f_atoms, amsg, woat, womt, bo, mask):
    return pl.pallas_call(
        _tc_out_body,
        grid=(N_ATOMS // BA,),
        in_specs=[
            pl.BlockSpec((BA, ATOM_FDIM), lambda i: (i, 0)),
            pl.BlockSpec((BA, HIDDEN), lambda i: (i, 0)),
            pl.BlockSpec((ATOM_FDIM, HIDDEN), lambda i: (0, 0)),
            pl.BlockSpec((HIDDEN, HIDDEN), lambda i: (0, 0)),
            pl.BlockSpec((1, HIDDEN), lambda i: (0, 0)),
            pl.BlockSpec((BA, 1), lambda i: (i, 0)),
        ],
        out_specs=pl.BlockSpec((BA, HIDDEN), lambda i: (i, 0)),
        out_shape=jax.ShapeDtypeStruct((N_ATOMS, HIDDEN), jnp.float32),
    )(f_atoms, amsg, woat, womt, bo, mask)


def kernel(f_atoms, f_bonds, a2b, b2a, b2revb, undirected_b2a, directed_b2a,
           parity_atoms, mask, W_i, W_ih, W_hh, b_ih, b_hh, W_o, b_o):
    wit = W_i.T
    wiht = W_ih.T
    whht = W_hh.T
    woat = W_o[:, :ATOM_FDIM].T
    womt = W_o[:, ATOM_FDIM:].T
    bih = b_ih.reshape(1, 3 * HIDDEN)
    bhh = b_hh.reshape(1, 3 * HIDDEN)
    bo = b_o.reshape(1, HIDDEN)
    a2b_flat = a2b.reshape(-1)

    inp = _tc_pre(f_bonds, wit)
    message = inp
    for _ in range(DEPTH - 1):
        amsg = _sc_gather_sum(message, a2b_flat)
        m_in = _sc_gather_diff(amsg, message, b2a, b2revb)
        message = _tc_gru(m_in, inp, wiht, bih, whht, bhh)
    amsg = _sc_gather_sum(message, a2b_flat)
    return _tc_out(f_atoms, amsg, woat, womt, bo, mask)


# speedup vs baseline: 1.1091x; 1.1091x over previous
import functools

import jax
import jax.numpy as jnp
from jax import lax
from jax.experimental import pallas as pl
from jax.experimental.pallas import tpu as pltpu
from jax.experimental.pallas import tpu_sc as plsc

N_ATOMS = 10000
N_BONDS = 320000
MAX_NB = 32
ATOM_FDIM = 128
BOND_FDIM = 16
HIDDEN = 128
DEPTH = 4

NC, NS, NL = 2, 16, 16
NW = NC * NS
NVH = HIDDEN // NL

_SC_MESH = plsc.VectorSubcoreMesh(core_axis_name="c", subcore_axis_name="s")

CA = 4
N_CHUNKS_A = N_ATOMS // CA
CI_A = CA * MAX_NB
CM_A = N_CHUNKS_A // NW
EXTRA_A = NW * CM_A


def _sum_chunk(rows_v, out_slab, o_base):
    for a in range(CA):
        accs = tuple(jnp.zeros((NL,), jnp.float32) for _ in range(NVH))

        def red(k2, acc, a=a):
            k = k2 * 4
            for u in range(4):
                acc = tuple(
                    acc[j] + rows_v[a * MAX_NB + k + u, pl.ds(j * NL, NL)]
                    for j in range(NVH))
            return acc

        accs = lax.fori_loop(0, MAX_NB // 4, red, accs)
        for j in range(NVH):
            out_slab[o_base + a, pl.ds(j * NL, NL)] = accs[j]


def _sc_gather_sum_body(msg_hbm, a2b_hbm, out_hbm, idx_v, rows0, rows1,
                        out_slab, xi_v, xo_v, sem0, sem1):
    wid = lax.axis_index("s") * NC + lax.axis_index("c")
    base_c = wid * CM_A
    base_idx = base_c * CI_A

    pltpu.sync_copy(a2b_hbm.at[pl.ds(base_idx, CM_A * CI_A)], idx_v)

    def g(i, rows, sem):
        return pltpu.make_async_copy(
            msg_hbm.at[idx_v.at[pl.ds(i * CI_A, CI_A)]], rows, sem)

    g(0, rows0, sem0).start()

    def pair(t, carry):
        i0 = 2 * t
        g(i0 + 1, rows1, sem1).start()
        g(i0, rows0, sem0).wait()
        _sum_chunk(rows0, out_slab, i0 * CA)

        @pl.when(t < CM_A // 2 - 1)
        def _():
            g(i0 + 2, rows0, sem0).start()

        g(i0 + 1, rows1, sem1).wait()
        _sum_chunk(rows1, out_slab, (i0 + 1) * CA)
        return carry

    lax.fori_loop(0, CM_A // 2, pair, 0)
    pltpu.sync_copy(out_slab, out_hbm.at[pl.ds(base_c * CA, CM_A * CA)])

    @pl.when(wid < N_CHUNKS_A - EXTRA_A)
    def _tail():
        c = EXTRA_A + wid
        pltpu.sync_copy(a2b_hbm.at[pl.ds(c * CI_A, CI_A)], xi_v)
        pltpu.async_copy(msg_hbm.at[xi_v], rows0, sem0).wait()
        _sum_chunk(rows0, xo_v, 0)
        pltpu.sync_copy(xo_v, out_hbm.at[pl.ds(c * CA, CA)])


_sc_gather_sum = functools.partial(
    pl.kernel,
    mesh=_SC_MESH,
    out_type=jax.ShapeDtypeStruct((N_ATOMS, HIDDEN), jnp.float32),
    scratch_types=[
        pltpu.VMEM((CM_A * CI_A,), jnp.int32),
        pltpu.VMEM((CI_A, HIDDEN), jnp.float32),
        pltpu.VMEM((CI_A, HIDDEN), jnp.float32),
        pltpu.VMEM((CM_A * CA, HIDDEN), jnp.float32),
        pltpu.VMEM((CI_A,), jnp.int32),
        pltpu.VMEM((CA, HIDDEN), jnp.float32),
        pltpu.SemaphoreType.DMA,
        pltpu.SemaphoreType.DMA,
    ],
)(_sc_gather_sum_body)


CB = 128
N_CHUNKS_B = N_BONDS // CB
CM_B = N_CHUNKS_B // NW
EXTRA_B = NW * CM_B


def _sub_chunk(rows_a, rows_r, obuf):
    def sub4(q, carry):
        b = q * 4
        for bb in range(4):
            for j in range(NVH):
                obuf[b + bb, pl.ds(j * NL, NL)] = (
                    rows_a[b + bb, pl.ds(j * NL, NL)]
                    - rows_r[b + bb, pl.ds(j * NL, NL)])
        return carry

    lax.fori_loop(0, CB // 4, sub4, 0)


def _sc_gather_diff_body(amsg_hbm, msg_hbm, b2a_hbm, b2revb_hbm, out_hbm,
                         idxa_v, idxr_v, a0, r0, a1, r1, o0, o1, xi_v,
                         sga0, sgr0, sga1, sgr1, so0, so1):
    wid = lax.axis_index("s") * NC + lax.axis_index("c")
    base_c = wid * CM_B
    base_row = base_c * CB

    pltpu.sync_copy(b2a_hbm.at[pl.ds(base_row, CM_B * CB)], idxa_v)
    pltpu.sync_copy(b2revb_hbm.at[pl.ds(base_row, CM_B * CB)], idxr_v)

    def ga(i, rows, sem):
        return pltpu.make_async_copy(
            amsg_hbm.at[idxa_v.at[pl.ds(i * CB, CB)]], rows, sem)

    def gr(i, rows, sem):
        return pltpu.make_async_copy(
            msg_hbm.at[idxr_v.at[pl.ds(i * CB, CB)]], rows, sem)

    def oc(i, obuf, sem):
        return pltpu.make_async_copy(
            obuf, out_hbm.at[pl.ds(base_row + i * CB, CB)], sem)

    ga(0, a0, sga0).start()
    gr(0, r0, sgr0).start()

    def pair(t, carry):
        i0 = 2 * t
        ga(i0 + 1, a1, sga1).start()
        gr(i0 + 1, r1, sgr1).start()
        ga(i0, a0, sga0).wait()
        gr(i0, r0, sgr0).wait()

        @pl.when(t > 0)
        def _():
            oc(i0 - 2, o0, so0).wait()

        _sub_chunk(a0, r0, o0)
        oc(i0, o0, so0).start()

        @pl.when(t < CM_B // 2 - 1)
        def _():
            ga(i0 + 2, a0, sga0).start()
            gr(i0 + 2, r0, sgr0).start()

        ga(i0 + 1, a1, sga1).wait()
        gr(i0 + 1, r1, sgr1).wait()

        @pl.when(t > 0)
        def _():
            oc(i0 - 1, o1, so1).wait()

        _sub_chunk(a1, r1, o1)
        oc(i0 + 1, o1, so1).start()
        return carry

    lax.fori_loop(0, CM_B // 2, pair, 0)
    oc(CM_B - 2, o0, so0).wait()
    oc(CM_B - 1, o1, so1).wait()

    @pl.when(wid < N_CHUNKS_B - EXTRA_B)
    def _tail():
        c = EXTRA_B + wid
        pltpu.sync_copy(b2a_hbm.at[pl.ds(c * CB, CB)], xi_v)
        pltpu.async_copy(amsg_hbm.at[xi_v], a0, sga0).wait()
        pltpu.sync_copy(b2revb_hbm.at[pl.ds(c * CB, CB)], xi_v)
        pltpu.async_copy(msg_hbm.at[xi_v], r0, sgr0).wait()
        _sub_chunk(a0, r0, o0)
        pltpu.sync_copy(o0, out_hbm.at[pl.ds(c * CB, CB)])


_sc_gather_diff = functools.partial(
    pl.kernel,
    mesh=_SC_MESH,
    out_type=jax.ShapeDtypeStruct((N_BONDS, HIDDEN), jnp.float32),
    scratch_types=[
        pltpu.VMEM((CM_B * CB,), jnp.int32),
        pltpu.VMEM((CM_B * CB,), jnp.int32),
        pltpu.VMEM((CB, HIDDEN), jnp.float32),
        pltpu.VMEM((CB, HIDDEN), jnp.float32),
        pltpu.VMEM((CB, HIDDEN), jnp.float32),
        pltpu.VMEM((CB, HIDDEN), jnp.float32),
        pltpu.VMEM((CB, HIDDEN), jnp.float32),
        pltpu.VMEM((CB, HIDDEN), jnp.float32),
        pltpu.VMEM((CB,), jnp.int32),
        pltpu.SemaphoreType.DMA,
        pltpu.SemaphoreType.DMA,
        pltpu.SemaphoreType.DMA,
        pltpu.SemaphoreType.DMA,
        pltpu.SemaphoreType.DMA,
        pltpu.SemaphoreType.DMA,
    ],
)(_sc_gather_diff_body)


BN = 4000
BA = 2000


def _tc_pre_body(fb_ref, wit_ref, inp_ref):
    inp_ref[...] = jnp.dot(fb_ref[...], wit_ref[...],
                           preferred_element_type=jnp.float32)


def _tc_pre(f_bonds, wit):
    return pl.pallas_call(
        _tc_pre_body,
        grid=(N_BONDS // BN,),
        in_specs=[
            pl.BlockSpec((BN, BOND_FDIM), lambda i: (i, 0)),
            pl.BlockSpec((BOND_FDIM, HIDDEN), lambda i: (0, 0)),
        ],
        out_specs=pl.BlockSpec((BN, HIDDEN), lambda i: (i, 0)),
        out_shape=jax.ShapeDtypeStruct((N_BONDS, HIDDEN), jnp.float32),
    )(f_bonds, wit)


def _tc_gru_body(m_ref, inp_ref, wiht_ref, bih_ref, whht_ref, bhh_ref,
                 out_ref):
    m = m_ref[...]
    gh = (jnp.dot(m, whht_ref[...], preferred_element_type=jnp.float32)
          + bhh_ref[...])
    gi = (jnp.dot(inp_ref[...], wiht_ref[...],
                  preferred_element_type=jnp.float32)
          + bih_ref[...])
    r = jax.nn.sigmoid(gi[:, :HIDDEN] + gh[:, :HIDDEN])
    z = jax.nn.sigmoid(gi[:, HIDDEN:2 * HIDDEN] + gh[:, HIDDEN:2 * HIDDEN])
    n = jnp.tanh(gi[:, 2 * HIDDEN:] + r * gh[:, 2 * HIDDEN:])
    out_ref[...] = (1.0 - z) * n + z * m

    @pl.when(pl.program_id(0) == 0)
    def _zero_row0():
        out_ref[0:1, :] = jnp.zeros((1, HIDDEN), jnp.float32)


def _tc_gru(m_in, inp, wiht, bih, whht, bhh):
    return pl.pallas_call(
        _tc_gru_body,
        grid=(N_BONDS // BN,),
        in_specs=[
            pl.BlockSpec((BN, HIDDEN), lambda i: (i, 0)),
            pl.BlockSpec((BN, HIDDEN), lambda i: (i, 0)),
            pl.BlockSpec((HIDDEN, 3 * HIDDEN), lambda i: (0, 0)),
            pl.BlockSpec((1, 3 * HIDDEN), lambda i: (0, 0)),
            pl.BlockSpec((HIDDEN, 3 * HIDDEN), lambda i: (0, 0)),
            pl.BlockSpec((1, 3 * HIDDEN), lambda i: (0, 0)),
        ],
        out_specs=pl.BlockSpec((BN, HIDDEN), lambda i: (i, 0)),
        out_shape=jax.ShapeDtypeStruct((N_BONDS, HIDDEN), jnp.float32),
    )(m_in, inp, wiht, bih, whht, bhh)


def _tc_out_body(fa_ref, am_ref, woa_ref, wom_ref, bo_ref, mask_ref, o_ref):
    h = (jnp.dot(fa_ref[...], woa_ref[...], preferred_element_type=jnp.float32)
         + jnp.dot(am_ref[...], wom_ref[...], preferred_element_type=jnp.float32)
         + bo_ref[...])
    o_ref[...] = jnp.maximum(h, 0.0) * mask_ref[...]


def _tc_out(f_atoms, amsg, woat, womt, bo, mask):
    return pl.pallas_call(
        _tc_out_body,
        grid=(N_ATOMS // BA,),
        in_specs=[
            pl.BlockSpec((BA, ATOM_FDIM), lambda i: (i, 0)),
            pl.BlockSpec((BA, HIDDEN), lambda i: (i, 0)),
            pl.BlockSpec((ATOM_FDIM, HIDDEN), lambda i: (0, 0)),
            pl.BlockSpec((HIDDEN, HIDDEN), lambda i: (0, 0)),
            pl.BlockSpec((1, HIDDEN), lambda i: (0, 0)),
            pl.BlockSpec((BA, 1), lambda i: (i, 0)),
        ],
        out_specs=pl.BlockSpec((BA, HIDDEN), lambda i: (i, 0)),
        out_shape=jax.ShapeDtypeStruct((N_ATOMS, HIDDEN), jnp.float32),
    )(f_atoms, amsg, woat, womt, bo, mask)


def kernel(f_atoms, f_bonds, a2b, b2a, b2revb, undirected_b2a, directed_b2a,
           parity_atoms, mask, W_i, W_ih, W_hh, b_ih, b_hh, W_o, b_o):
    wit = W_i.T
    wiht = W_ih.T
    whht = W_hh.T
    woat = W_o[:, :ATOM_FDIM].T
    womt = W_o[:, ATOM_FDIM:].T
    bih = b_ih.reshape(1, 3 * HIDDEN)
    bhh = b_hh.reshape(1, 3 * HIDDEN)
    bo = b_o.reshape(1, HIDDEN)
    a2b_flat = a2b.reshape(-1)

    inp = _tc_pre(f_bonds, wit)
    message = inp
    for _ in range(DEPTH - 1):
        amsg = _sc_gather_sum(message, a2b_flat)
        m_in = _sc_gather_diff(amsg, message, b2a, b2revb)
        message = _tc_gru(m_in, inp, wiht, bih, whht, bhh)
    amsg = _sc_gather_sum(message, a2b_flat)
    return _tc_out(f_atoms, amsg, woat, womt, bo, mask)


# speedup vs baseline: 1.1553x; 1.0417x over previous
import functools

import jax
import jax.numpy as jnp
from jax import lax
from jax.experimental import pallas as pl
from jax.experimental.pallas import tpu as pltpu
from jax.experimental.pallas import tpu_sc as plsc

N_ATOMS = 10000
N_BONDS = 320000
MAX_NB = 32
ATOM_FDIM = 128
BOND_FDIM = 16
HIDDEN = 128
DEPTH = 4

NC, NS, NL = 2, 16, 16
NW = NC * NS
NVH = HIDDEN // NL

_SC_MESH = plsc.VectorSubcoreMesh(core_axis_name="c", subcore_axis_name="s")

CA = 4
N_CHUNKS_A = N_ATOMS // CA
CI_A = CA * MAX_NB
CM_A = N_CHUNKS_A // NW
EXTRA_A = NW * CM_A


def _sum_chunk(rows_v, out_slab, o_base):
    for a in range(CA):
        accs = tuple(jnp.zeros((NL,), jnp.float32) for _ in range(NVH))

        def red(k2, acc, a=a):
            k = k2 * 4
            for u in range(4):
                acc = tuple(
                    acc[j] + rows_v[a * MAX_NB + k + u, pl.ds(j * NL, NL)]
                    for j in range(NVH))
            return acc

        accs = lax.fori_loop(0, MAX_NB // 4, red, accs)
        for j in range(NVH):
            out_slab[o_base + a, pl.ds(j * NL, NL)] = accs[j]


def _sc_gather_sum_body(msg_hbm, a2b_hbm, out_hbm, idx_v, rows0, rows1,
                        out_slab, xi_v, xo_v, sem0, sem1):
    wid = lax.axis_index("s") * NC + lax.axis_index("c")
    base_c = wid * CM_A
    base_idx = base_c * CI_A

    pltpu.sync_copy(a2b_hbm.at[pl.ds(base_idx, CM_A * CI_A)], idx_v)

    def g(i, rows, sem):
        return pltpu.make_async_copy(
            msg_hbm.at[idx_v.at[pl.ds(i * CI_A, CI_A)]], rows, sem)

    g(0, rows0, sem0).start()

    def pair(t, carry):
        i0 = 2 * t
        g(i0 + 1, rows1, sem1).start()
        g(i0, rows0, sem0).wait()
        _sum_chunk(rows0, out_slab, i0 * CA)

        @pl.when(t < CM_A // 2 - 1)
        def _():
            g(i0 + 2, rows0, sem0).start()

        g(i0 + 1, rows1, sem1).wait()
        _sum_chunk(rows1, out_slab, (i0 + 1) * CA)
        return carry

    lax.fori_loop(0, CM_A // 2, pair, 0)
    pltpu.sync_copy(out_slab, out_hbm.at[pl.ds(base_c * CA, CM_A * CA)])

    @pl.when(wid < N_CHUNKS_A - EXTRA_A)
    def _tail():
        c = EXTRA_A + wid
        pltpu.sync_copy(a2b_hbm.at[pl.ds(c * CI_A, CI_A)], xi_v)
        pltpu.async_copy(msg_hbm.at[xi_v], rows0, sem0).wait()
        _sum_chunk(rows0, xo_v, 0)
        pltpu.sync_copy(xo_v, out_hbm.at[pl.ds(c * CA, CA)])


_sc_gather_sum = functools.partial(
    pl.kernel,
    mesh=_SC_MESH,
    out_type=jax.ShapeDtypeStruct((N_ATOMS, HIDDEN), jnp.float32),
    scratch_types=[
        pltpu.VMEM((CM_A * CI_A,), jnp.int32),
        pltpu.VMEM((CI_A, HIDDEN), jnp.float32),
        pltpu.VMEM((CI_A, HIDDEN), jnp.float32),
        pltpu.VMEM((CM_A * CA, HIDDEN), jnp.float32),
        pltpu.VMEM((CI_A,), jnp.int32),
        pltpu.VMEM((CA, HIDDEN), jnp.float32),
        pltpu.SemaphoreType.DMA,
        pltpu.SemaphoreType.DMA,
    ],
)(_sc_gather_sum_body)


CB = 128
N_CHUNKS_B = N_BONDS // CB
CM_B = N_CHUNKS_B // NW
EXTRA_B = NW * CM_B


def _sub_chunk(rows_a, rows_r, obuf):
    def sub4(q, carry):
        b = q * 4
        for bb in range(4):
            for j in range(NVH):
                obuf[b + bb, pl.ds(j * NL, NL)] = (
                    rows_a[b + bb, pl.ds(j * NL, NL)]
                    - rows_r[b + bb, pl.ds(j * NL, NL)])
        return carry

    lax.fori_loop(0, CB // 4, sub4, 0)


def _sc_gather_diff_body(amsg_hbm, msg_hbm, b2a_hbm, b2revb_hbm, out_hbm,
                         idxa_v, idxr_v, a0, r0, a1, r1, o0, o1, xi_v,
                         sga0, sgr0, sga1, sgr1, so0, so1):
    wid = lax.axis_index("s") * NC + lax.axis_index("c")
    base_c = wid * CM_B
    base_row = base_c * CB

    pltpu.sync_copy(b2a_hbm.at[pl.ds(base_row, CM_B * CB)], idxa_v)
    pltpu.sync_copy(b2revb_hbm.at[pl.ds(base_row, CM_B * CB)], idxr_v)

    def ga(i, rows, sem):
        return pltpu.make_async_copy(
            amsg_hbm.at[idxa_v.at[pl.ds(i * CB, CB)]], rows, sem)

    def gr(i, rows, sem):
        return pltpu.make_async_copy(
            msg_hbm.at[idxr_v.at[pl.ds(i * CB, CB)]], rows, sem)

    def oc(i, obuf, sem):
        return pltpu.make_async_copy(
            obuf, out_hbm.at[pl.ds(base_row + i * CB, CB)], sem)

    ga(0, a0, sga0).start()
    gr(0, r0, sgr0).start()

    def pair(t, carry):
        i0 = 2 * t
        ga(i0 + 1, a1, sga1).start()
        gr(i0 + 1, r1, sgr1).start()
        ga(i0, a0, sga0).wait()
        gr(i0, r0, sgr0).wait()

        @pl.when(t > 0)
        def _():
            oc(i0 - 2, o0, so0).wait()

        _sub_chunk(a0, r0, o0)
        oc(i0, o0, so0).start()

        @pl.when(t < CM_B // 2 - 1)
        def _():
            ga(i0 + 2, a0, sga0).start()
            gr(i0 + 2, r0, sgr0).start()

        ga(i0 + 1, a1, sga1).wait()
        gr(i0 + 1, r1, sgr1).wait()

        @pl.when(t > 0)
        def _():
            oc(i0 - 1, o1, so1).wait()

        _sub_chunk(a1, r1, o1)
        oc(i0 + 1, o1, so1).start()
        return carry

    lax.fori_loop(0, CM_B // 2, pair, 0)
    oc(CM_B - 2, o0, so0).wait()
    oc(CM_B - 1, o1, so1).wait()

    @pl.when(wid < N_CHUNKS_B - EXTRA_B)
    def _tail():
        c = EXTRA_B + wid
        pltpu.sync_copy(b2a_hbm.at[pl.ds(c * CB, CB)], xi_v)
        pltpu.async_copy(amsg_hbm.at[xi_v], a0, sga0).wait()
        pltpu.sync_copy(b2revb_hbm.at[pl.ds(c * CB, CB)], xi_v)
        pltpu.async_copy(msg_hbm.at[xi_v], r0, sgr0).wait()
        _sub_chunk(a0, r0, o0)
        pltpu.sync_copy(o0, out_hbm.at[pl.ds(c * CB, CB)])


_sc_gather_diff = functools.partial(
    pl.kernel,
    mesh=_SC_MESH,
    out_type=jax.ShapeDtypeStruct((N_BONDS, HIDDEN), jnp.float32),
    scratch_types=[
        pltpu.VMEM((CM_B * CB,), jnp.int32),
        pltpu.VMEM((CM_B * CB,), jnp.int32),
        pltpu.VMEM((CB, HIDDEN), jnp.float32),
        pltpu.VMEM((CB, HIDDEN), jnp.float32),
        pltpu.VMEM((CB, HIDDEN), jnp.float32),
        pltpu.VMEM((CB, HIDDEN), jnp.float32),
        pltpu.VMEM((CB, HIDDEN), jnp.float32),
        pltpu.VMEM((CB, HIDDEN), jnp.float32),
        pltpu.VMEM((CB,), jnp.int32),
        pltpu.SemaphoreType.DMA,
        pltpu.SemaphoreType.DMA,
        pltpu.SemaphoreType.DMA,
        pltpu.SemaphoreType.DMA,
        pltpu.SemaphoreType.DMA,
        pltpu.SemaphoreType.DMA,
    ],
)(_sc_gather_diff_body)


BN = 8000
BA = 2000


def _tc_pre_body(fb_ref, wit_ref, inp_ref):
    inp_ref[...] = jnp.dot(fb_ref[...], wit_ref[...],
                           preferred_element_type=jnp.float32)


def _tc_pre(f_bonds, wit):
    return pl.pallas_call(
        _tc_pre_body,
        grid=(N_BONDS // BN,),
        in_specs=[
            pl.BlockSpec((BN, BOND_FDIM), lambda i: (i, 0)),
            pl.BlockSpec((BOND_FDIM, HIDDEN), lambda i: (0, 0)),
        ],
        out_specs=pl.BlockSpec((BN, HIDDEN), lambda i: (i, 0)),
        out_shape=jax.ShapeDtypeStruct((N_BONDS, HIDDEN), jnp.float32),
    )(f_bonds, wit)


def _tc_gru_body(m_ref, inp_ref, wiht_ref, bih_ref, whht_ref, bhh_ref,
                 out_ref):
    m = m_ref[...]
    gh = (jnp.dot(m, whht_ref[...], preferred_element_type=jnp.float32)
          + bhh_ref[...])
    gi = (jnp.dot(inp_ref[...], wiht_ref[...],
                  preferred_element_type=jnp.float32)
          + bih_ref[...])
    r = jax.nn.sigmoid(gi[:, :HIDDEN] + gh[:, :HIDDEN])
    z = jax.nn.sigmoid(gi[:, HIDDEN:2 * HIDDEN] + gh[:, HIDDEN:2 * HIDDEN])
    n = jnp.tanh(gi[:, 2 * HIDDEN:] + r * gh[:, 2 * HIDDEN:])
    out_ref[...] = (1.0 - z) * n + z * m

    @pl.when(pl.program_id(0) == 0)
    def _zero_row0():
        out_ref[0:1, :] = jnp.zeros((1, HIDDEN), jnp.float32)


def _tc_gru(m_in, inp, wiht, bih, whht, bhh):
    return pl.pallas_call(
        _tc_gru_body,
        grid=(N_BONDS // BN,),
        in_specs=[
            pl.BlockSpec((BN, HIDDEN), lambda i: (i, 0)),
            pl.BlockSpec((BN, HIDDEN), lambda i: (i, 0)),
            pl.BlockSpec((HIDDEN, 3 * HIDDEN), lambda i: (0, 0)),
            pl.BlockSpec((1, 3 * HIDDEN), lambda i: (0, 0)),
            pl.BlockSpec((HIDDEN, 3 * HIDDEN), lambda i: (0, 0)),
            pl.BlockSpec((1, 3 * HIDDEN), lambda i: (0, 0)),
        ],
        out_specs=pl.BlockSpec((BN, HIDDEN), lambda i: (i, 0)),
        out_shape=jax.ShapeDtypeStruct((N_BONDS, HIDDEN), jnp.float32),
    )(m_in, inp, wiht, bih, whht, bhh)


def _tc_out_body(fa_ref, am_ref, woa_ref, wom_ref, bo_ref, mask_ref, o_ref):
    h = (jnp.dot(fa_ref[...], woa_ref[...], preferred_element_type=jnp.float32)
         + jnp.dot(am_ref[...], wom_ref[...], preferred_element_type=jnp.float32)
         + bo_ref[...])
    o_ref[...] = jnp.maximum(h, 0.0) * mask_ref[...]


def _tc_out(f_atoms, amsg, woat, womt, bo, mask):
    return pl.pallas_call(
        _tc_out_body,
        grid=(N_ATOMS // BA,),
        in_specs=[
            pl.BlockSpec((BA, ATOM_FDIM), lambda i: (i, 0)),
            pl.BlockSpec((BA, HIDDEN), lambda i: (i, 0)),
            pl.BlockSpec((ATOM_FDIM, HIDDEN), lambda i: (0, 0)),
            pl.BlockSpec((HIDDEN, HIDDEN), lambda i: (0, 0)),
            pl.BlockSpec((1, HIDDEN), lambda i: (0, 0)),
            pl.BlockSpec((BA, 1), lambda i: (i, 0)),
        ],
        out_specs=pl.BlockSpec((BA, HIDDEN), lambda i: (i, 0)),
        out_shape=jax.ShapeDtypeStruct((N_ATOMS, HIDDEN), jnp.float32),
    )(f_atoms, amsg, woat, womt, bo, mask)


def kernel(f_atoms, f_bonds, a2b, b2a, b2revb, undirected_b2a, directed_b2a,
           parity_atoms, mask, W_i, W_ih, W_hh, b_ih, b_hh, W_o, b_o):
    wit = W_i.T
    wiht = W_ih.T
    whht = W_hh.T
    woat = W_o[:, :ATOM_FDIM].T
    womt = W_o[:, ATOM_FDIM:].T
    bih = b_ih.reshape(1, 3 * HIDDEN)
    bhh = b_hh.reshape(1, 3 * HIDDEN)
    bo = b_o.reshape(1, HIDDEN)
    a2b_flat = a2b.reshape(-1)

    inp = _tc_pre(f_bonds, wit)
    message = inp
    for _ in range(DEPTH - 1):
        amsg = _sc_gather_sum(message, a2b_flat)
        m_in = _sc_gather_diff(amsg, message, b2a, b2revb)
        message = _tc_gru(m_in, inp, wiht, bih, whht, bhh)
    amsg = _sc_gather_sum(message, a2b_flat)
    return _tc_out(f_atoms, amsg, woat, womt, bo, mask)
